# baseline (speedup 1.0000x reference)
import jax
import jax.numpy as jnp
from jax.experimental import pallas as pl

_EPS = 1e-5


def _c1_body(pats_ref, w_ref, prm_ref, out_ref):
    A = pats_ref[0]
    y = jax.lax.dot_general(A, w_ref[...], (((0,), (1,)), ((), ())),
                            preferred_element_type=jnp.float32)
    b1 = prm_ref[0:1, :]
    g = prm_ref[1:2, :]
    be = prm_ref[2:3, :]
    rm = prm_ref[3:4, :]
    rv = prm_ref[4:5, :]
    y = y + b1
    y = (y - rm) / jnp.sqrt(rv + _EPS) * g + be
    out_ref[0] = jnp.maximum(y, 0.0)


def _c2_body(pats_ref, y1_ref, w_ref, prm_ref, out_ref):
    A = pats_ref[0]
    y = jax.lax.dot_general(A, w_ref[...], (((0,), (1,)), ((), ())),
                            preferred_element_type=jnp.float32)
    b2 = prm_ref[0:1, :]
    g = prm_ref[1:2, :]
    be = prm_ref[2:3, :]
    rm = prm_ref[3:4, :]
    rv = prm_ref[4:5, :]
    y = (y + b2) + y1_ref[0]
    y = (y - rm) / jnp.sqrt(rv + _EPS) * g + be
    out_ref[0] = y


def _vq_body(y_ref, cb_ref, cbt_ref, lat_ref, idx_ref, part_ref):
    Y = y_ref[0]
    cb = cb_ref[...]
    y2 = jnp.sum(Y * Y, axis=1, keepdims=True)
    c2 = jnp.sum(cb * cb, axis=1)
    E = jax.lax.dot_general(Y, cb, (((1,), (1,)), ((), ())),
                            preferred_element_type=jnp.float32)
    dist = (y2 - 2.0 * E) + c2[None, :]
    m = jnp.min(dist, axis=1, keepdims=True)
    K = 1024
    iota1 = jax.lax.broadcasted_iota(jnp.int32, (1024, 1024), 1)
    idx = jnp.min(jnp.where(dist == m, iota1, K), axis=1, keepdims=True)
    idx_row = idx.reshape(1, 1024)
    iota0 = jax.lax.broadcasted_iota(jnp.int32, (1024, 1024), 0)
    onehot = (iota0 == idx_row).astype(jnp.float32)
    latT = jax.lax.dot_general(cbt_ref[...], onehot, (((1,), (0,)), ((), ())),
                               preferred_element_type=jnp.float32,
                               precision=jax.lax.Precision.HIGHEST)
    lat_ref[...] = latT[None]
    idx_ref[...] = idx_row[None]
    part_ref[...] = jnp.full((1, 1, 128), jnp.sum(m), jnp.float32)


def kernel(x, conv1_w, conv1_b, bn1_g, bn1_b, bn1_rm, bn1_rv, res_w, res_b, bn2_g, bn2_b, bn2_rm, bn2_rv, codebook):
    B = x.shape[0]
    xq = jnp.transpose(x.reshape(B, 2, 256, 2, 256, 2), (0, 1, 3, 5, 2, 4))
    xqp = jnp.pad(xq, ((0, 0), (0, 0), (0, 0), (0, 0), (1, 1), (1, 1)))
    rmap = [(1, -1), (0, 0), (1, 0), (0, 1)]
    p1 = []
    for ci in range(2):
        for kh in range(4):
            pr, dr = rmap[kh]
            for kw in range(4):
                pc, dc = rmap[kw]
                p1.append(xqp[:, ci, pr, pc, 1 + dr:257 + dr, 1 + dc:257 + dc])
    pats1 = jnp.stack(p1, axis=1).astype(jnp.bfloat16).reshape(B, 32, 65536)
    w1 = conv1_w.reshape(4, 32).astype(jnp.bfloat16)
    prm1 = jnp.stack([conv1_b, bn1_g, bn1_b, bn1_rm, bn1_rv], axis=0)

    y1pix = pl.pallas_call(
        _c1_body,
        grid=(B, 8),
        in_specs=[
            pl.BlockSpec((1, 32, 8192), lambda b, m: (b, 0, m)),
            pl.BlockSpec((4, 32), lambda b, m: (0, 0)),
            pl.BlockSpec((5, 4), lambda b, m: (0, 0)),
        ],
        out_specs=pl.BlockSpec((1, 8192, 4), lambda b, m: (b, m, 0)),
        out_shape=jax.ShapeDtypeStruct((B, 65536, 4), jnp.float32),
    )(pats1, w1, prm1)

    y1pl = jnp.transpose(y1pix.reshape(B, 256, 256, 4), (0, 3, 1, 2))
    ypad = jnp.pad(y1pl, ((0, 0), (0, 0), (1, 1), (1, 1)))
    p2 = []
    for ci in range(4):
        for kh in range(3):
            for kw in range(3):
                p2.append(ypad[:, ci, kh:kh + 256, kw:kw + 256])
    pats2 = jnp.stack(p2, axis=1).astype(jnp.bfloat16).reshape(B, 36, 65536)
    w2 = res_w.reshape(4, 36).astype(jnp.bfloat16)
    prm2 = jnp.stack([res_b, bn2_g, bn2_b, bn2_rm, bn2_rv], axis=0)

    y2pix = pl.pallas_call(
        _c2_body,
        grid=(B, 8),
        in_specs=[
            pl.BlockSpec((1, 36, 8192), lambda b, m: (b, 0, m)),
            pl.BlockSpec((1, 8192, 4), lambda b, m: (b, m, 0)),
            pl.BlockSpec((4, 36), lambda b, m: (0, 0)),
            pl.BlockSpec((5, 4), lambda b, m: (0, 0)),
        ],
        out_specs=pl.BlockSpec((1, 8192, 4), lambda b, m: (b, m, 0)),
        out_shape=jax.ShapeDtypeStruct((B, 65536, 4), jnp.float32),
    )(pats2, y1pix, w2, prm2)

    yt = jnp.transpose(y2pix.reshape(B, 256, 256, 4), (0, 3, 2, 1)).reshape(B, 1024, 256)
    cbt = jnp.transpose(codebook, (1, 0))

    lat, idx, part = pl.pallas_call(
        _vq_body,
        grid=(B,),
        in_specs=[
            pl.BlockSpec((1, 1024, 256), lambda b: (b, 0, 0)),
            pl.BlockSpec((1024, 256), lambda b: (0, 0)),
            pl.BlockSpec((256, 1024), lambda b: (0, 0)),
        ],
        out_specs=[
            pl.BlockSpec((1, 256, 1024), lambda b: (b, 0, 0)),
            pl.BlockSpec((1, 1, 1024), lambda b: (b, 0, 0)),
            pl.BlockSpec((1, 1, 128), lambda b: (b, 0, 0)),
        ],
        out_shape=[
            jax.ShapeDtypeStruct((B, 256, 1024), jnp.float32),
            jax.ShapeDtypeStruct((B, 1, 1024), jnp.int32),
            jax.ShapeDtypeStruct((B, 1, 128), jnp.float32),
        ],
    )(yt, codebook, cbt)

    latent = lat
    indices = idx.reshape(B, 1024)[..., None]
    commit_loss = (0.01 * (jnp.sum(part[:, 0, 0]) / (B * 1024 * 256)))[None]
    return latent, indices, commit_loss


# speedup vs baseline: 1.4133x; 1.4133x over previous
import jax
import jax.numpy as jnp
from jax.experimental import pallas as pl

_EPS = 1e-5


def _c1_body(pats_ref, w_ref, prm_ref, out_ref):
    A = pats_ref[0]
    y = jax.lax.dot_general(A, w_ref[...], (((0,), (1,)), ((), ())),
                            preferred_element_type=jnp.float32)
    b1 = prm_ref[0:1, :]
    g = prm_ref[1:2, :]
    be = prm_ref[2:3, :]
    rm = prm_ref[3:4, :]
    rv = prm_ref[4:5, :]
    y = y + b1
    y = (y - rm) / jnp.sqrt(rv + _EPS) * g + be
    out_ref[0] = jnp.maximum(y, 0.0)


def _c2_body(pats_ref, y1_ref, w_ref, prm_ref, out_ref):
    A = pats_ref[0]
    y = jax.lax.dot_general(A, w_ref[...], (((0,), (1,)), ((), ())),
                            preferred_element_type=jnp.float32)
    b2 = prm_ref[0:1, :]
    g = prm_ref[1:2, :]
    be = prm_ref[2:3, :]
    rm = prm_ref[3:4, :]
    rv = prm_ref[4:5, :]
    y = (y + b2) + y1_ref[0]
    y = (y - rm) / jnp.sqrt(rv + _EPS) * g + be
    out_ref[0] = y


def _vq_body(y_ref, cb_ref, cbt_ref, lat_ref, idx_ref, part_ref):
    Y = y_ref[0]
    cb = cb_ref[...]
    y2 = jnp.sum(Y * Y, axis=1, keepdims=True)
    c2 = jnp.sum(cb * cb, axis=1)
    E = jax.lax.dot_general(Y, cb, (((1,), (1,)), ((), ())),
                            preferred_element_type=jnp.float32)
    dist = (y2 - 2.0 * E) + c2[None, :]
    m = jnp.min(dist, axis=1, keepdims=True)
    K = 1024
    iota1 = jax.lax.broadcasted_iota(jnp.int32, (1024, 1024), 1)
    idx = jnp.min(jnp.where(dist == m, iota1, K), axis=1, keepdims=True)
    idx_row = idx.reshape(1, 1024)
    iota0 = jax.lax.broadcasted_iota(jnp.int32, (1024, 1024), 0)
    onehot = (iota0 == idx_row).astype(jnp.float32)
    latT = jax.lax.dot_general(cbt_ref[...], onehot, (((1,), (0,)), ((), ())),
                               preferred_element_type=jnp.float32,
                               precision=jax.lax.Precision.HIGHEST)
    lat_ref[...] = latT[None]
    idx_ref[...] = idx_row[None]
    part_ref[...] = jnp.full((1, 1, 128), jnp.sum(m), jnp.float32)


def kernel(x, conv1_w, conv1_b, bn1_g, bn1_b, bn1_rm, bn1_rv, res_w, res_b, bn2_g, bn2_b, bn2_rm, bn2_rv, codebook):
    B = x.shape[0]
    xpad = jnp.pad(x, ((0, 0), (0, 0), (1, 1), (1, 1)))
    p1 = []
    for ci in range(2):
        for kh in range(4):
            for kw in range(4):
                p1.append(xpad[:, ci, kh:kh + 512:2, kw:kw + 512:2])
    pats1 = jnp.stack(p1, axis=1).astype(jnp.bfloat16).reshape(B, 32, 65536)
    w1 = conv1_w.reshape(4, 32).astype(jnp.bfloat16)
    prm1 = jnp.stack([conv1_b, bn1_g, bn1_b, bn1_rm, bn1_rv], axis=0)

    y1pix = pl.pallas_call(
        _c1_body,
        grid=(B, 8),
        in_specs=[
            pl.BlockSpec((1, 32, 8192), lambda b, m: (b, 0, m)),
            pl.BlockSpec((4, 32), lambda b, m: (0, 0)),
            pl.BlockSpec((5, 4), lambda b, m: (0, 0)),
        ],
        out_specs=pl.BlockSpec((1, 8192, 4), lambda b, m: (b, m, 0)),
        out_shape=jax.ShapeDtypeStruct((B, 65536, 4), jnp.float32),
    )(pats1, w1, prm1)

    y1pl = jnp.transpose(y1pix.reshape(B, 256, 256, 4), (0, 3, 1, 2))
    ypad = jnp.pad(y1pl, ((0, 0), (0, 0), (1, 1), (1, 1)))
    p2 = []
    for ci in range(4):
        for kh in range(3):
            for kw in range(3):
                p2.append(ypad[:, ci, kh:kh + 256, kw:kw + 256])
    pats2 = jnp.stack(p2, axis=1).astype(jnp.bfloat16).reshape(B, 36, 65536)
    w2 = res_w.reshape(4, 36).astype(jnp.bfloat16)
    prm2 = jnp.stack([res_b, bn2_g, bn2_b, bn2_rm, bn2_rv], axis=0)

    y2pix = pl.pallas_call(
        _c2_body,
        grid=(B, 8),
        in_specs=[
            pl.BlockSpec((1, 36, 8192), lambda b, m: (b, 0, m)),
            pl.BlockSpec((1, 8192, 4), lambda b, m: (b, m, 0)),
            pl.BlockSpec((4, 36), lambda b, m: (0, 0)),
            pl.BlockSpec((5, 4), lambda b, m: (0, 0)),
        ],
        out_specs=pl.BlockSpec((1, 8192, 4), lambda b, m: (b, m, 0)),
        out_shape=jax.ShapeDtypeStruct((B, 65536, 4), jnp.float32),
    )(pats2, y1pix, w2, prm2)

    yt = jnp.transpose(y2pix.reshape(B, 256, 256, 4), (0, 3, 2, 1)).reshape(B, 1024, 256)
    cbt = jnp.transpose(codebook, (1, 0))

    lat, idx, part = pl.pallas_call(
        _vq_body,
        grid=(B,),
        in_specs=[
            pl.BlockSpec((1, 1024, 256), lambda b: (b, 0, 0)),
            pl.BlockSpec((1024, 256), lambda b: (0, 0)),
            pl.BlockSpec((256, 1024), lambda b: (0, 0)),
        ],
        out_specs=[
            pl.BlockSpec((1, 256, 1024), lambda b: (b, 0, 0)),
            pl.BlockSpec((1, 1, 1024), lambda b: (b, 0, 0)),
            pl.BlockSpec((1, 1, 128), lambda b: (b, 0, 0)),
        ],
        out_shape=[
            jax.ShapeDtypeStruct((B, 256, 1024), jnp.float32),
            jax.ShapeDtypeStruct((B, 1, 1024), jnp.int32),
            jax.ShapeDtypeStruct((B, 1, 128), jnp.float32),
        ],
    )(yt, codebook, cbt)

    latent = lat
    indices = idx.reshape(B, 1024)[..., None]
    commit_loss = (0.01 * (jnp.sum(part[:, 0, 0]) / (B * 1024 * 256)))[None]
    return latent, indices, commit_loss


# speedup vs baseline: 1.5273x; 1.0807x over previous
import jax
import jax.numpy as jnp
from jax.experimental import pallas as pl

_EPS = 1e-5


def _c1_body(pats_ref, w_ref, prm_ref, out_ref):
    A = pats_ref[0]
    y = jax.lax.dot_general(A, w_ref[...], (((0,), (1,)), ((), ())),
                            preferred_element_type=jnp.float32)
    yT = jnp.transpose(y, (1, 0))
    b1 = prm_ref[:, 0:1]
    g = prm_ref[:, 1:2]
    be = prm_ref[:, 2:3]
    rm = prm_ref[:, 3:4]
    rv = prm_ref[:, 4:5]
    yT = yT + b1
    yT = (yT - rm) / jnp.sqrt(rv + _EPS) * g + be
    out_ref[0] = jnp.maximum(yT, 0.0)


def _c2_body(pats_ref, y1_ref, w_ref, prm_ref, out_ref):
    A = pats_ref[0]
    y = jax.lax.dot_general(A, w_ref[...], (((0,), (1,)), ((), ())),
                            preferred_element_type=jnp.float32)
    yT = jnp.transpose(y, (1, 0))
    b2 = prm_ref[:, 0:1]
    g = prm_ref[:, 1:2]
    be = prm_ref[:, 2:3]
    rm = prm_ref[:, 3:4]
    rv = prm_ref[:, 4:5]
    yT = (yT + b2) + y1_ref[0]
    yT = (yT - rm) / jnp.sqrt(rv + _EPS) * g + be
    out_ref[0] = yT


def _vq_body(y_ref, cb_ref, cbt_ref, lat_ref, idx_ref, part_ref):
    Y = y_ref[0]
    cb = cb_ref[...]
    y2 = jnp.sum(Y * Y, axis=1, keepdims=True)
    c2 = jnp.sum(cb * cb, axis=1)
    E = jax.lax.dot_general(Y, cb, (((1,), (1,)), ((), ())),
                            preferred_element_type=jnp.float32)
    dist = (y2 - 2.0 * E) + c2[None, :]
    m = jnp.min(dist, axis=1, keepdims=True)
    K = 1024
    iota1 = jax.lax.broadcasted_iota(jnp.int32, (1024, 1024), 1)
    idx = jnp.min(jnp.where(dist == m, iota1, K), axis=1, keepdims=True)
    idx_row = idx.reshape(1, 1024)
    iota0 = jax.lax.broadcasted_iota(jnp.int32, (1024, 1024), 0)
    onehot = (iota0 == idx_row).astype(jnp.float32)
    latT = jax.lax.dot_general(cbt_ref[...], onehot, (((1,), (0,)), ((), ())),
                               preferred_element_type=jnp.float32,
                               precision=jax.lax.Precision.HIGHEST)
    lat_ref[...] = latT[None]
    idx_ref[...] = idx_row[None]
    part_ref[...] = jnp.full((1, 1, 128), jnp.sum(m), jnp.float32)


def kernel(x, conv1_w, conv1_b, bn1_g, bn1_b, bn1_rm, bn1_rv, res_w, res_b, bn2_g, bn2_b, bn2_rm, bn2_rv, codebook):
    B = x.shape[0]
    xpad = jnp.pad(x, ((0, 0), (0, 0), (1, 1), (1, 1)))
    p1 = []
    for ci in range(2):
        for kh in range(4):
            for kw in range(4):
                p1.append(xpad[:, ci, kh:kh + 512:2, kw:kw + 512:2])
    pats1 = jnp.stack(p1, axis=1).astype(jnp.bfloat16).reshape(B, 32, 65536)
    w1 = conv1_w.reshape(4, 32).astype(jnp.bfloat16)
    prm1 = jnp.stack([conv1_b, bn1_g, bn1_b, bn1_rm, bn1_rv], axis=1)

    y1pix = pl.pallas_call(
        _c1_body,
        grid=(B, 8),
        in_specs=[
            pl.BlockSpec((1, 32, 8192), lambda b, m: (b, 0, m)),
            pl.BlockSpec((4, 32), lambda b, m: (0, 0)),
            pl.BlockSpec((4, 5), lambda b, m: (0, 0)),
        ],
        out_specs=pl.BlockSpec((1, 4, 8192), lambda b, m: (b, 0, m)),
        out_shape=jax.ShapeDtypeStruct((B, 4, 65536), jnp.float32),
    )(pats1, w1, prm1)

    y1pl = y1pix.reshape(B, 4, 256, 256)
    ypad = jnp.pad(y1pl, ((0, 0), (0, 0), (1, 1), (1, 1)))
    p2 = []
    for ci in range(4):
        for kh in range(3):
            for kw in range(3):
                p2.append(ypad[:, ci, kh:kh + 256, kw:kw + 256])
    pats2 = jnp.stack(p2, axis=1).astype(jnp.bfloat16).reshape(B, 36, 65536)
    w2 = res_w.reshape(4, 36).astype(jnp.bfloat16)
    prm2 = jnp.stack([res_b, bn2_g, bn2_b, bn2_rm, bn2_rv], axis=1)

    y2pix = pl.pallas_call(
        _c2_body,
        grid=(B, 8),
        in_specs=[
            pl.BlockSpec((1, 36, 8192), lambda b, m: (b, 0, m)),
            pl.BlockSpec((1, 4, 8192), lambda b, m: (b, 0, m)),
            pl.BlockSpec((4, 36), lambda b, m: (0, 0)),
            pl.BlockSpec((4, 5), lambda b, m: (0, 0)),
        ],
        out_specs=pl.BlockSpec((1, 4, 8192), lambda b, m: (b, 0, m)),
        out_shape=jax.ShapeDtypeStruct((B, 4, 65536), jnp.float32),
    )(pats2, y1pl.reshape(B, 4, 65536), w2, prm2)

    yt = jnp.transpose(y2pix.reshape(B, 4, 256, 256), (0, 1, 3, 2)).reshape(B, 1024, 256)
    cbt = jnp.transpose(codebook, (1, 0))

    lat, idx, part = pl.pallas_call(
        _vq_body,
        grid=(B,),
        in_specs=[
            pl.BlockSpec((1, 1024, 256), lambda b: (b, 0, 0)),
            pl.BlockSpec((1024, 256), lambda b: (0, 0)),
            pl.BlockSpec((256, 1024), lambda b: (0, 0)),
        ],
        out_specs=[
            pl.BlockSpec((1, 256, 1024), lambda b: (b, 0, 0)),
            pl.BlockSpec((1, 1, 1024), lambda b: (b, 0, 0)),
            pl.BlockSpec((1, 1, 128), lambda b: (b, 0, 0)),
        ],
        out_shape=[
            jax.ShapeDtypeStruct((B, 256, 1024), jnp.float32),
            jax.ShapeDtypeStruct((B, 1, 1024), jnp.int32),
            jax.ShapeDtypeStruct((B, 1, 128), jnp.float32),
        ],
    )(yt, codebook, cbt)

    latent = lat
    indices = idx.reshape(B, 1024)[..., None]
    commit_loss = (0.01 * (jnp.sum(part[:, 0, 0]) / (B * 1024 * 256)))[None]
    return latent, indices, commit_loss
